# baseline (speedup 1.0000x reference)
import functools

import jax
import jax.numpy as jnp
from jax.experimental import pallas as pl

_TOP_K = 2
_N_EXPERTS = 8
_N_GROUP = 4
_TOPK_GROUP = 2


def _route_rows(s_rows):
    f32 = s_rows[0].dtype
    i32 = jnp.int32
    g = [jnp.maximum(s_rows[2 * k], s_rows[2 * k + 1]) for k in range(4)]
    m1 = jnp.maximum(jnp.maximum(g[0], g[1]), jnp.maximum(g[2], g[3]))
    gi1 = jnp.where(
        g[0] == m1, 0,
        jnp.where(g[1] == m1, 1, jnp.where(g[2] == m1, 2, 3))).astype(i32)
    ge = [jnp.where(gi1 == k, jnp.asarray(-1.0, f32), g[k]) for k in range(4)]
    m2 = jnp.maximum(jnp.maximum(ge[0], ge[1]), jnp.maximum(ge[2], ge[3]))
    gi2 = jnp.where(
        ge[0] == m2, 0,
        jnp.where(ge[1] == m2, 1, jnp.where(ge[2] == m2, 2, 3))).astype(i32)
    ms = [
        jnp.where((gi1 == (e // 2)) | (gi2 == (e // 2)), s_rows[e],
                  jnp.asarray(0.0, f32)) for e in range(8)
    ]
    M1 = ms[0]
    for e in range(1, 8):
        M1 = jnp.maximum(M1, ms[e])
    e1 = jnp.asarray(7, i32)
    for e in range(6, -1, -1):
        e1 = jnp.where(ms[e] == M1, e, e1).astype(i32)
    mse = [jnp.where(e1 == e, jnp.asarray(-1.0, f32), ms[e]) for e in range(8)]
    M2 = mse[0]
    for e in range(1, 8):
        M2 = jnp.maximum(M2, mse[e])
    e2 = jnp.asarray(7, i32)
    for e in range(6, -1, -1):
        e2 = jnp.where(mse[e] == M2, e, e2).astype(i32)
    denom = M1 + M2 + jnp.asarray(1e-20, f32)
    return e1, e2, M1 / denom, M2 / denom


def _gate_block(x_ref, w_ref, idx_ref, wgt_ref):
    x = x_ref[...]
    w = w_ref[...]
    lt = jax.lax.dot_general(w, x, (((1,), (1,)), ((), ())),
                             preferred_element_type=jnp.float32)
    m = jnp.max(lt, axis=0, keepdims=True)
    ex = jnp.exp(lt - m)
    s = ex / jnp.sum(ex, axis=0, keepdims=True)
    rows = [s[e:e + 1, :] for e in range(8)]
    e1, e2, w1, w2 = _route_rows(rows)
    idx_ref[...] = jnp.concatenate([e1, e2], axis=0)
    wgt_ref[...] = jnp.concatenate([w1, w2], axis=0)


@functools.partial(jax.jit, static_argnames=("block_t",))
def _moe_gate_tc(x, weight, block_t=2048):
    t = x.shape[0]
    grid = (t // block_t,)
    idx_t, wgt_t = pl.pallas_call(
        _gate_block,
        grid=grid,
        in_specs=[
            pl.BlockSpec((block_t, x.shape[1]), lambda i: (i, 0)),
            pl.BlockSpec((weight.shape[0], weight.shape[1]), lambda i: (0, 0)),
        ],
        out_specs=[
            pl.BlockSpec((2, block_t), lambda i: (0, i)),
            pl.BlockSpec((2, block_t), lambda i: (0, i)),
        ],
        out_shape=[
            jax.ShapeDtypeStruct((2, t), jnp.int32),
            jax.ShapeDtypeStruct((2, t), jnp.float32),
        ],
    )(x, weight)
    return idx_t.T, wgt_t.T


def kernel(hidden_states, weight):
    bsz, seq_len, h = hidden_states.shape
    x = hidden_states.reshape(-1, h)
    topk_idx, topk_weight = _moe_gate_tc(x, weight)
    return topk_idx, topk_weight


# speedup vs baseline: 1.0280x; 1.0280x over previous
import functools

import jax
import jax.numpy as jnp
from jax.experimental import pallas as pl

_TOP_K = 2
_N_EXPERTS = 8
_N_GROUP = 4
_TOPK_GROUP = 2


def _route_rows(s_rows):
    f32 = s_rows[0].dtype
    i32 = jnp.int32
    g = [jnp.maximum(s_rows[2 * k], s_rows[2 * k + 1]) for k in range(4)]
    m1 = jnp.maximum(jnp.maximum(g[0], g[1]), jnp.maximum(g[2], g[3]))
    gi1 = jnp.where(
        g[0] == m1, 0,
        jnp.where(g[1] == m1, 1, jnp.where(g[2] == m1, 2, 3))).astype(i32)
    ge = [jnp.where(gi1 == k, jnp.asarray(-1.0, f32), g[k]) for k in range(4)]
    m2 = jnp.maximum(jnp.maximum(ge[0], ge[1]), jnp.maximum(ge[2], ge[3]))
    gi2 = jnp.where(
        ge[0] == m2, 0,
        jnp.where(ge[1] == m2, 1, jnp.where(ge[2] == m2, 2, 3))).astype(i32)
    ms = [
        jnp.where((gi1 == (e // 2)) | (gi2 == (e // 2)), s_rows[e],
                  jnp.asarray(0.0, f32)) for e in range(8)
    ]
    M1 = ms[0]
    for e in range(1, 8):
        M1 = jnp.maximum(M1, ms[e])
    e1 = jnp.asarray(7, i32)
    for e in range(6, -1, -1):
        e1 = jnp.where(ms[e] == M1, e, e1).astype(i32)
    mse = [jnp.where(e1 == e, jnp.asarray(-1.0, f32), ms[e]) for e in range(8)]
    M2 = mse[0]
    for e in range(1, 8):
        M2 = jnp.maximum(M2, mse[e])
    e2 = jnp.asarray(7, i32)
    for e in range(6, -1, -1):
        e2 = jnp.where(mse[e] == M2, e, e2).astype(i32)
    denom = M1 + M2 + jnp.asarray(1e-20, f32)
    return e1, e2, M1 / denom, M2 / denom


def _gate_block(x_ref, w_ref, idx_ref, wgt_ref):
    x = x_ref[...]
    w = w_ref[...]
    lt = jax.lax.dot_general(w, x, (((1,), (1,)), ((), ())),
                             preferred_element_type=jnp.float32)
    m = jnp.max(lt, axis=0, keepdims=True)
    ex = jnp.exp(lt - m)
    s = ex / jnp.sum(ex, axis=0, keepdims=True)
    rows = [s[e:e + 1, :] for e in range(8)]
    e1, e2, w1, w2 = _route_rows(rows)
    idx_ref[...] = jnp.concatenate([e1, e2], axis=0)
    wgt_ref[...] = jnp.concatenate([w1, w2], axis=0)


@functools.partial(jax.jit, static_argnames=("block_t",))
def _moe_gate_tc(x, weight, block_t=1024):
    t = x.shape[0]
    grid = (t // block_t,)
    idx_t, wgt_t = pl.pallas_call(
        _gate_block,
        grid=grid,
        in_specs=[
            pl.BlockSpec((block_t, x.shape[1]), lambda i: (i, 0)),
            pl.BlockSpec((weight.shape[0], weight.shape[1]), lambda i: (0, 0)),
        ],
        out_specs=[
            pl.BlockSpec((2, block_t), lambda i: (0, i)),
            pl.BlockSpec((2, block_t), lambda i: (0, i)),
        ],
        out_shape=[
            jax.ShapeDtypeStruct((2, t), jnp.int32),
            jax.ShapeDtypeStruct((2, t), jnp.float32),
        ],
    )(x, weight)
    return idx_t.T, wgt_t.T


def kernel(hidden_states, weight):
    bsz, seq_len, h = hidden_states.shape
    x = hidden_states.reshape(-1, h)
    topk_idx, topk_weight = _moe_gate_tc(x, weight)
    return topk_idx, topk_weight
